# baseline (speedup 1.0000x reference)
import dataclasses

import jax
import jax.numpy as jnp
from jax import lax
from jax.experimental import pallas as pl
from jax.experimental.pallas import tpu as pltpu
from jax.experimental.pallas import tpu_sc as plsc

N = 10000
E = 320000
D = 128

NC = 2
NS = 16
NW = NC * NS

K_CH = 64
E_PW = 10240
E_PAD = E_PW * NW
N_CHUNKS = E_PW // K_CH
NBUF = 3

TOT_CH = E_PAD // K_CH // NS
K0 = TOT_CH // 2
K1 = TOT_CH - K0

N_S = 10112
ROWS_PER_SUB = N_S // NS

DGR = 79
ZB = 64
Z_FULL = ROWS_PER_SUB // ZB
Z_REM = ROWS_PER_SUB % ZB

PREP_BLK = 1000
FIN_BLK = 2048
FIN_GRID = 5


def _prep_body(x_ref, wl_ref, wr_ref, b_ref, h_ref, r_ref):
    xb = x_ref[...]
    h_ref[...] = lax.dot_general(
        xb, wl_ref[...], (((1,), (1,)), ((), ())),
        preferred_element_type=jnp.float32,
        precision=lax.Precision.HIGHEST)
    r_ref[...] = lax.dot_general(
        xb, wr_ref[...], (((1,), (1,)), ((), ())),
        preferred_element_type=jnp.float32,
        precision=lax.Precision.HIGHEST) + b_ref[...]


def _tc_prep(x, W_l, W_r, b_l):
    return pl.pallas_call(
        _prep_body,
        grid=(N // PREP_BLK,),
        in_specs=[
            pl.BlockSpec((PREP_BLK, D), lambda i: (i, 0)),
            pl.BlockSpec((D, D), lambda i: (0, 0)),
            pl.BlockSpec((D, D), lambda i: (0, 0)),
            pl.BlockSpec((1, D), lambda i: (0, 0)),
        ],
        out_specs=[
            pl.BlockSpec((PREP_BLK, D), lambda i: (i, 0)),
            pl.BlockSpec((PREP_BLK, D), lambda i: (i, 0)),
        ],
        out_shape=[
            jax.ShapeDtypeStruct((N, D), jnp.float32),
            jax.ShapeDtypeStruct((N, D), jnp.float32),
        ],
    )(x, W_l, W_r, b_l)


GRP = 8


def _sc_body(src_hbm, dst_hbm, h_hbm, s_out, d_out,
             srcst, dstst, rows, deg, s_sh, gsems, ssems):
    cid = lax.axis_index("c")
    sid = lax.axis_index("s")

    zrows = NBUF * K_CH
    @pl.loop(0, zrows)
    def _(i):
        for j in range(D // 16):
            rows[i, pl.ds(j * 16, 16)] = jnp.zeros((16,), jnp.float32)

    @pl.loop(0, DGR)
    def _(i):
        for j in range(D // 16):
            deg[i, pl.ds(j * 16, 16)] = jnp.zeros((16,), jnp.float32)

    @pl.loop(0, ROWS_PER_SUB // zrows)
    def _(i):
        base = sid * ROWS_PER_SUB + i * zrows
        pltpu.sync_copy(rows, s_sh.at[pl.ds(base, zrows)])

    zrem = ROWS_PER_SUB % zrows
    if zrem:
        rbase = sid * ROWS_PER_SUB + (ROWS_PER_SUB // zrows) * zrows
        pltpu.sync_copy(rows.at[pl.ds(0, zrem)], s_sh.at[pl.ds(rbase, zrem)])

    plsc.subcore_barrier()

    wid = cid * NS + sid
    one16 = jnp.ones((16,), jnp.float32)

    def rbuf(j):
        b = j % NBUF
        return rows.at[pl.ds(b * K_CH, K_CH)], gsems.at[b], ssems.at[b]

    def run_groups(row0, ngroups):
        @pl.loop(0, ngroups)
        def _(g):
            gbase = row0 + g * GRP
            pltpu.sync_copy(src_hbm.at[pl.ds(gbase, GRP)], srcst)
            pltpu.sync_copy(dst_hbm.at[pl.ds(gbase, GRP)], dstst)

            for jj in range(2):
                buf, gsem, _ = rbuf(jj)
                pltpu.async_copy(h_hbm.at[srcst.at[jj]], buf, gsem)

            for jj in range(GRP):
                buf, gsem, ssem = rbuf(jj)
                if jj + 2 < GRP:
                    if jj - 2 >= 0:
                        pbuf, _, pssem = rbuf(jj - 2)
                        pltpu.make_async_copy(
                            pbuf, s_sh.at[dstst.at[jj - 2]], pssem).wait()
                    nb, ngsem, _ = rbuf(jj + 2)
                    pltpu.async_copy(h_hbm.at[srcst.at[jj + 2]], nb, ngsem)
                pltpu.make_async_copy(h_hbm.at[srcst.at[jj]], buf, gsem).wait()
                pltpu.async_copy(buf, s_sh.at[dstst.at[jj]], ssem, add=True)
                for k in range(K_CH // 16):
                    idx = dstst[jj, pl.ds(k * 16, 16)]
                    plsc.addupdate_scatter(
                        deg, [lax.shift_right_logical(idx, 7),
                              lax.bitwise_and(idx, 127)], one16)

            for jj in range(GRP - NBUF, GRP):
                buf, _, ssem = rbuf(jj)
                pltpu.make_async_copy(buf, s_sh.at[dstst.at[jj]], ssem).wait()

    if K0 > 0:
        @pl.when(cid == 0)
        def _():
            run_groups(sid * K0, K0 // GRP)

    if K1 > 0:
        @pl.when(cid == 1)
        def _():
            run_groups(NS * K0 + sid * K1, K1 // GRP)

    plsc.subcore_barrier()

    lo = sid * ROWS_PER_SUB
    pltpu.sync_copy(s_sh.at[pl.ds(lo, ROWS_PER_SUB)],
                    s_out.at[cid, pl.ds(lo, ROWS_PER_SUB)])
    pltpu.sync_copy(deg, d_out.at[wid])


def _sc_compiler_params():
    cp = pltpu.CompilerParams()
    if "needs_layout_passes" in pltpu.CompilerParams.__dataclass_fields__:
        cp = dataclasses.replace(cp, needs_layout_passes=False)
    return cp


def _sc_aggregate(src2d, dst2d, h):
    mesh = plsc.VectorSubcoreMesh(core_axis_name="c", subcore_axis_name="s")
    kfn = pl.kernel(
        _sc_body,
        out_type=(
            jax.ShapeDtypeStruct((NC, N_S, D), jnp.float32),
            jax.ShapeDtypeStruct((NW, DGR, D), jnp.float32),
        ),
        mesh=mesh,
        scratch_types=[
            pltpu.VMEM((GRP, K_CH), jnp.int32),
            pltpu.VMEM((GRP, K_CH), jnp.int32),
            pltpu.VMEM((NBUF * K_CH, D), jnp.float32),
            pltpu.VMEM((DGR, D), jnp.float32),
            pltpu.VMEM_SHARED((N_S, D), jnp.float32),
            pltpu.SemaphoreType.DMA((NBUF,)),
            pltpu.SemaphoreType.DMA((NBUF,)),
        ],
        compiler_params=_sc_compiler_params(),
    )
    return kfn(src2d, dst2d, h)


def _deg_body(d_ref, inv_ref):
    deg = jnp.sum(d_ref[...], axis=0)
    inv_ref[...] = 1.0 / jnp.maximum(deg, 1.0)


def _tc_degmerge(d_part):
    return pl.pallas_call(
        _deg_body,
        in_specs=[pl.BlockSpec((NW, DGR, D), lambda: (0, 0, 0))],
        out_specs=pl.BlockSpec((DGR, D), lambda: (0, 0)),
        out_shape=jax.ShapeDtypeStruct((DGR, D), jnp.float32),
    )(d_part)


def _final_body(s_ref, i_ref, r_ref, o_ref):
    s = s_ref[0] + s_ref[1]
    o_ref[...] = jnp.maximum(s * i_ref[...] + r_ref[...], 0.0)


def _tc_final(s_part, inv_col, r):
    return pl.pallas_call(
        _final_body,
        grid=(FIN_GRID,),
        in_specs=[
            pl.BlockSpec((NC, FIN_BLK, D), lambda i: (0, i, 0)),
            pl.BlockSpec((FIN_BLK, 1), lambda i: (i, 0)),
            pl.BlockSpec((FIN_BLK, D), lambda i: (i, 0)),
        ],
        out_specs=pl.BlockSpec((FIN_BLK, D), lambda i: (i, 0)),
        out_shape=jax.ShapeDtypeStruct((N, D), jnp.float32),
    )(s_part, inv_col, r)


@jax.jit
def kernel(x, edge_index, W_l, b_l, W_r):
    src = edge_index[0].astype(jnp.int32)
    dst = edge_index[1].astype(jnp.int32)
    pad = E_PAD - E
    src2d = jnp.concatenate(
        [src, jnp.zeros((pad,), jnp.int32)]).reshape(E_PAD // K_CH, K_CH)
    dst2d = jnp.concatenate(
        [dst, jnp.full((pad,), N, jnp.int32)]).reshape(E_PAD // K_CH, K_CH)

    h, r = _tc_prep(x, W_l, W_r, b_l.reshape(1, D))
    s_part, d_part = _sc_aggregate(src2d, dst2d, h)
    inv_col = _tc_degmerge(d_part).reshape(DGR * D, 1)
    return _tc_final(s_part, inv_col, r)


# speedup vs baseline: 1.0524x; 1.0524x over previous
import dataclasses

import jax
import jax.numpy as jnp
from jax import lax
from jax.experimental import pallas as pl
from jax.experimental.pallas import tpu as pltpu
from jax.experimental.pallas import tpu_sc as plsc

N = 10000
E = 320000
D = 128

NC = 2
NS = 16
NW = NC * NS

K_CH = 128
E_PW = 10240
E_PAD = E_PW * NW
N_CHUNKS = E_PW // K_CH

N_S = 10112
ROWS_PER_SUB = N_S // NS

DGR = 80
ZB = 64
Z_FULL = ROWS_PER_SUB // ZB
Z_REM = ROWS_PER_SUB % ZB

PREP_BLK = 1000
FIN_BLK = 2048
FIN_GRID = 5


def _prep_body(x_ref, wl_ref, wr_ref, b_ref, h_ref, r_ref):
    xb = x_ref[...]
    h_ref[...] = lax.dot_general(
        xb, wl_ref[...], (((1,), (1,)), ((), ())),
        preferred_element_type=jnp.float32,
        precision=lax.Precision.HIGHEST)
    r_ref[...] = lax.dot_general(
        xb, wr_ref[...], (((1,), (1,)), ((), ())),
        preferred_element_type=jnp.float32,
        precision=lax.Precision.HIGHEST) + b_ref[...]


def _tc_prep(x, W_l, W_r, b_l):
    return pl.pallas_call(
        _prep_body,
        grid=(N // PREP_BLK,),
        in_specs=[
            pl.BlockSpec((PREP_BLK, D), lambda i: (i, 0)),
            pl.BlockSpec((D, D), lambda i: (0, 0)),
            pl.BlockSpec((D, D), lambda i: (0, 0)),
            pl.BlockSpec((1, D), lambda i: (0, 0)),
        ],
        out_specs=[
            pl.BlockSpec((PREP_BLK, D), lambda i: (i, 0)),
            pl.BlockSpec((PREP_BLK, D), lambda i: (i, 0)),
        ],
        out_shape=[
            jax.ShapeDtypeStruct((N, D), jnp.float32),
            jax.ShapeDtypeStruct((N, D), jnp.float32),
        ],
    )(x, W_l, W_r, b_l)


GRP = 8


def _sc_body(src_hbm, dst_hbm, h_hbm, s_out,
             srcst, dstst, rows0, rows1, s_sh, sem0, sem1):
    cid = lax.axis_index("c")
    sid = lax.axis_index("s")

    @pl.loop(0, K_CH)
    def _(i):
        for j in range(D // 16):
            rows0[i, pl.ds(j * 16, 16)] = jnp.zeros((16,), jnp.float32)

    @pl.loop(0, ROWS_PER_SUB // K_CH)
    def _(i):
        base = sid * ROWS_PER_SUB + i * K_CH
        pltpu.sync_copy(rows0, s_sh.at[pl.ds(base, K_CH)])

    zrem = ROWS_PER_SUB % K_CH
    if zrem:
        rbase = sid * ROWS_PER_SUB + (ROWS_PER_SUB // K_CH) * K_CH
        pltpu.sync_copy(rows0.at[pl.ds(0, zrem)], s_sh.at[pl.ds(rbase, zrem)])

    plsc.subcore_barrier()

    wid = cid * NS + sid
    row0 = wid * N_CHUNKS

    @pl.loop(0, N_CHUNKS // GRP)
    def _(g):
        gbase = row0 + g * GRP
        pltpu.sync_copy(src_hbm.at[pl.ds(gbase, GRP)], srcst)
        pltpu.sync_copy(dst_hbm.at[pl.ds(gbase, GRP)], dstst)

        pltpu.async_copy(h_hbm.at[srcst.at[0]], rows0, sem0)
        pltpu.async_copy(h_hbm.at[srcst.at[1]], rows1, sem1)

        @pl.loop(0, GRP, step=2)
        def _(j):
            for b, (rbuf, sem) in enumerate(((rows0, sem0), (rows1, sem1))):
                jj = j + b
                pltpu.make_async_copy(h_hbm.at[srcst.at[jj]], rbuf, sem).wait()
                pltpu.sync_copy(rbuf, s_sh.at[dstst.at[jj]], add=True)

                @pl.when(jj + 2 < GRP)
                def _():
                    pltpu.async_copy(h_hbm.at[srcst.at[jj + 2]], rbuf, sem)

    plsc.subcore_barrier()

    lo = sid * ROWS_PER_SUB
    pltpu.sync_copy(s_sh.at[pl.ds(lo, ROWS_PER_SUB)],
                    s_out.at[cid, pl.ds(lo, ROWS_PER_SUB)])


def _deg_sc_body(dst_hbm, d_out, dstall, deg):
    cid = lax.axis_index("c")
    sid = lax.axis_index("s")
    wid = cid * NS + sid
    one16 = jnp.ones((16,), jnp.float32)

    @pl.loop(0, DGR)
    def _(i):
        for j in range(D // 16):
            deg[i, pl.ds(j * 16, 16)] = jnp.zeros((16,), jnp.float32)

    pltpu.sync_copy(dst_hbm.at[pl.ds(wid * N_CHUNKS, N_CHUNKS)], dstall)

    @pl.loop(0, N_CHUNKS)
    def _(i):
        for k in range(K_CH // 16):
            idx = dstall[i, pl.ds(k * 16, 16)]
            plsc.addupdate_scatter(
                deg, [lax.shift_right_logical(idx, 7),
                      lax.bitwise_and(idx, 127)], one16)

    pltpu.sync_copy(deg, d_out.at[wid])


def _sc_compiler_params():
    cp = pltpu.CompilerParams()
    if "needs_layout_passes" in pltpu.CompilerParams.__dataclass_fields__:
        cp = dataclasses.replace(cp, needs_layout_passes=False)
    return cp


def _sc_aggregate(src2d, dst2d, h):
    mesh = plsc.VectorSubcoreMesh(core_axis_name="c", subcore_axis_name="s")
    kfn = pl.kernel(
        _sc_body,
        out_type=jax.ShapeDtypeStruct((NC, N_S, D), jnp.float32),
        mesh=mesh,
        scratch_types=[
            pltpu.VMEM((GRP, 128), jnp.int32),
            pltpu.VMEM((GRP, 128), jnp.int32),
            pltpu.VMEM((K_CH, D), jnp.float32),
            pltpu.VMEM((K_CH, D), jnp.float32),
            pltpu.VMEM_SHARED((N_S, D), jnp.float32),
            pltpu.SemaphoreType.DMA,
            pltpu.SemaphoreType.DMA,
        ],
        compiler_params=_sc_compiler_params(),
    )
    return kfn(src2d, dst2d, h)


def _sc_degree(dst2d):
    mesh = plsc.VectorSubcoreMesh(core_axis_name="c", subcore_axis_name="s")
    kfn = pl.kernel(
        _deg_sc_body,
        out_type=jax.ShapeDtypeStruct((NW, DGR, D), jnp.float32),
        mesh=mesh,
        scratch_types=[
            pltpu.VMEM((N_CHUNKS, 128), jnp.int32),
            pltpu.VMEM((DGR, D), jnp.float32),
        ],
        compiler_params=_sc_compiler_params(),
    )
    return kfn(dst2d)


def _deg_body(d_ref, inv_ref):
    deg = jnp.sum(d_ref[...], axis=0)
    inv_ref[...] = 1.0 / jnp.maximum(deg, 1.0)


def _tc_degmerge(d_part):
    return pl.pallas_call(
        _deg_body,
        in_specs=[pl.BlockSpec((NW, DGR, D), lambda: (0, 0, 0))],
        out_specs=pl.BlockSpec((DGR, D), lambda: (0, 0)),
        out_shape=jax.ShapeDtypeStruct((DGR, D), jnp.float32),
    )(d_part)


def _final_body(s_ref, i_ref, r_ref, o_ref):
    s = s_ref[0] + s_ref[1]
    o_ref[...] = jnp.maximum(s * i_ref[...] + r_ref[...], 0.0)


def _tc_final(s_part, inv_col, r):
    return pl.pallas_call(
        _final_body,
        grid=(FIN_GRID,),
        in_specs=[
            pl.BlockSpec((NC, FIN_BLK, D), lambda i: (0, i, 0)),
            pl.BlockSpec((FIN_BLK, 1), lambda i: (i, 0)),
            pl.BlockSpec((FIN_BLK, D), lambda i: (i, 0)),
        ],
        out_specs=pl.BlockSpec((FIN_BLK, D), lambda i: (i, 0)),
        out_shape=jax.ShapeDtypeStruct((N, D), jnp.float32),
    )(s_part, inv_col, r)


@jax.jit
def kernel(x, edge_index, W_l, b_l, W_r):
    src = edge_index[0].astype(jnp.int32)
    dst = edge_index[1].astype(jnp.int32)
    pad = E_PAD - E
    src2d = jnp.concatenate(
        [src, jnp.zeros((pad,), jnp.int32)]).reshape(E_PAD // 128, 128)
    dst2d = jnp.concatenate(
        [dst, jnp.full((pad,), N, jnp.int32)]).reshape(E_PAD // 128, 128)

    d_part = _sc_degree(dst2d)
    h, r = _tc_prep(x, W_l, W_r, b_l.reshape(1, D))
    s_part = _sc_aggregate(src2d, dst2d, h)
    inv_col = _tc_degmerge(d_part).reshape(DGR * D, 1)
    return _tc_final(s_part, inv_col, r)
